# baseline (speedup 1.0000x reference)
import functools

import jax
import jax.numpy as jnp
from jax import lax
from jax.experimental import pallas as pl
from jax.experimental.pallas import tpu as pltpu
from jax.experimental.pallas import tpu_sc as plsc

NUM_USERS = 1000000
NUM_ITEMS = 1000000
EMB = 32
B, L = 4096, 50
N = B * L
CH = 128
NC, NS = 2, 16
NW = NC * NS
CPT = N // (NW * CH)
PR = 16
KG = 10
CROWS = 1024
N_BLK = N // CROWS
TBW = 8192


def _scatter_pos_body(item_hbm, pval_hbm, pos_hbm, idx_v, src_v, sem):
    wid = lax.axis_index("s") * NC + lax.axis_index("c")
    pltpu.sync_copy(item_hbm.at[wid], idx_v)

    def group(g, carry):
        pltpu.sync_copy(pval_hbm.at[wid, pl.ds(g * KG, KG)], src_v)
        for j in range(KG):
            pltpu.async_copy(
                src_v.at[j], pos_hbm.at[idx_v.at[g * KG + j]], sem
            )
        for j in range(KG):
            pltpu.make_async_copy(
                src_v.at[j], pos_hbm.at[idx_v.at[g * KG + j]], sem
            ).wait()
        return carry

    lax.fori_loop(0, CPT // KG, group, 0)


def _gather_body(user_hbm, item_hbm, item16_hbm, pos_hbm, users_hbm, ic_hbm,
                 it_hbm, ue_hbm, ice_hbm, ite_hbm, q_hbm,
                 uidx_v, iidx_v, i16_v, ubuf, cbuf, tbuf, qacc_v, gsem):
    wid = lax.axis_index("s") * NC + lax.axis_index("c")
    base = wid * CPT
    pltpu.sync_copy(user_hbm.at[wid], uidx_v)
    pltpu.sync_copy(item_hbm.at[wid], iidx_v)
    pltpu.sync_copy(item16_hbm.at[wid], i16_v)

    def chunk(j, carry):
        cu = pltpu.async_copy(users_hbm.at[uidx_v.at[j]], ubuf, gsem)
        cc = pltpu.async_copy(ic_hbm.at[iidx_v.at[j]], cbuf, gsem)
        ct = pltpu.async_copy(it_hbm.at[iidx_v.at[j]], tbuf, gsem)
        cq = pltpu.async_copy(
            pos_hbm.at[i16_v.at[j]], qacc_v.at[pl.ds(j * CH, CH)], gsem
        )
        cu.wait()
        cc.wait()
        ct.wait()
        cq.wait()
        p0 = (base + j) * CH
        b = p0 // CROWS
        rr0 = p0 % CROWS
        pltpu.sync_copy(ubuf, ue_hbm.at[b, pl.ds(rr0, CH), pl.ds(0, EMB)])
        pltpu.sync_copy(cbuf, ice_hbm.at[b, pl.ds(rr0, CH), pl.ds(0, EMB)])
        pltpu.sync_copy(tbuf, ite_hbm.at[b, pl.ds(rr0, CH), pl.ds(0, EMB)])
        return carry

    lax.fori_loop(0, CPT, chunk, 0)
    pltpu.sync_copy(qacc_v, q_hbm.at[pl.ds(wid * CPT * CH, CPT * CH)])


@functools.cache
def _sc_kernels():
    mesh = plsc.VectorSubcoreMesh(
        core_axis_name="c", subcore_axis_name="s", num_cores=NC, num_subcores=NS
    )
    params = pltpu.CompilerParams(use_tc_tiling_on_sc=False)
    scatter_pos = pl.kernel(
        _scatter_pos_body,
        out_type=jax.ShapeDtypeStruct((NUM_ITEMS, PR), jnp.int32),
        mesh=mesh,
        scratch_types=[
            pltpu.VMEM((CPT, CH), jnp.int32),
            pltpu.VMEM((KG, CH, PR), jnp.int32),
            pltpu.SemaphoreType.DMA,
        ],
        name="sc_scatter_pos",
        compiler_params=params,
    )
    gather = pl.kernel(
        _gather_body,
        out_type=(
            jax.ShapeDtypeStruct((N_BLK, CROWS, 128), jnp.float32),
            jax.ShapeDtypeStruct((N_BLK, CROWS, 128), jnp.float32),
            jax.ShapeDtypeStruct((N_BLK, CROWS, 128), jnp.float32),
            jax.ShapeDtypeStruct((N,), jnp.int32),
        ),
        mesh=mesh,
        scratch_types=[
            pltpu.VMEM((CPT, CH), jnp.int32),
            pltpu.VMEM((CPT, CH), jnp.int32),
            pltpu.VMEM((CPT, CH), jnp.int32),
            pltpu.VMEM((CH, EMB), jnp.float32),
            pltpu.VMEM((CH, EMB), jnp.float32),
            pltpu.VMEM((CH, EMB), jnp.float32),
            pltpu.VMEM((CPT * CH,), jnp.int32),
            pltpu.SemaphoreType.DMA,
        ],
        name="sc_gather",
        compiler_params=params,
    )
    return scatter_pos, gather


def _transpose_body(ut_ref, ct_ref, tt_ref, u_out, c_out, t_out):
    u_out[...] = ut_ref[...].T
    c_out[...] = ct_ref[...].T
    t_out[...] = tt_ref[...].T


_N_TBLK = -(-NUM_ITEMS // TBW)
_transpose = pl.pallas_call(
    _transpose_body,
    grid=(_N_TBLK,),
    in_specs=[pl.BlockSpec((EMB, TBW), lambda i: (0, i))] * 3,
    out_specs=[pl.BlockSpec((TBW, EMB), lambda i: (i, 0))] * 3,
    out_shape=[jax.ShapeDtypeStruct((NUM_ITEMS, EMB), jnp.float32)] * 3,
)


def _compute_body(ue_ref, ice_ref, ite_ref, lab_ref, w_ref, q_ref, *outs):
    i = pl.program_id(0)
    u = ue_ref[0]
    c = ice_ref[0]
    t = ite_ref[0]
    lab = lab_ref[0]
    w = w_ref[0]
    q = q_ref[0]

    valid = (
        lax.broadcasted_iota(jnp.int32, (CROWS, 128), 1) < EMB
    )
    ones = jnp.ones((128, 1), jnp.float32)

    def rowsum_t(x):
        xz = jnp.where(valid, x, 0.0)
        return lax.dot_general(ones, xz, (((0,), (1,)), ((), ())))

    sc = rowsum_t(u * c)
    st = rowsum_t(u * t)
    d = c - t
    s = rowsum_t(d * d)

    pidx = i * CROWS + lax.broadcasted_iota(jnp.int32, (1, CROWS), 1)
    winf = (q == pidx).astype(jnp.float32)
    nw = 1.0 - w

    def bce(x):
        return jnp.maximum(x, 0.0) - x * lab + jnp.log1p(jnp.exp(-jnp.abs(x)))

    sig = lambda x: 1.0 / (1.0 + jnp.exp(-x))
    sums = (
        jnp.sum(bce(sc) * nw),
        jnp.sum(bce(st) * w),
        jnp.sum(jnp.abs(sig(sc) - lab) * nw),
        jnp.sum(jnp.abs(sig(st) - lab) * w),
        jnp.sum(w),
        jnp.sum(s * winf),
        jnp.sum(winf),
    )
    for o_ref, val in zip(outs, sums):
        @pl.when(i == 0)
        def _init(o_ref=o_ref):
            o_ref[...] = jnp.zeros_like(o_ref)

        o_ref[...] += val


_N_SUMS = 7
_compute = pl.pallas_call(
    _compute_body,
    grid=(N_BLK,),
    in_specs=[
        pl.BlockSpec((1, CROWS, 128), lambda i: (i, 0, 0)),
        pl.BlockSpec((1, CROWS, 128), lambda i: (i, 0, 0)),
        pl.BlockSpec((1, CROWS, 128), lambda i: (i, 0, 0)),
        pl.BlockSpec((1, 1, CROWS), lambda i: (i, 0, 0)),
        pl.BlockSpec((1, 1, CROWS), lambda i: (i, 0, 0)),
        pl.BlockSpec((1, 1, CROWS), lambda i: (i, 0, 0)),
    ],
    out_specs=[pl.BlockSpec((1, 128), lambda i: (0, 0))] * _N_SUMS,
    out_shape=[jax.ShapeDtypeStruct((1, 128), jnp.float32)] * _N_SUMS,
)


def kernel(user, item, label, mask, users, items_control, items_treatment):
    user3d = user.reshape(NW, CPT, CH)
    item3d = item.reshape(NW, CPT, CH)
    pval4d = jnp.broadcast_to(
        jnp.arange(N, dtype=jnp.int32)[:, None], (N, PR)
    ).reshape(NW, CPT, CH, PR)
    item16_3d = (item * PR).reshape(NW, CPT, CH)

    users_rm, ic_rm, it_rm = _transpose(
        users.T, items_control.T, items_treatment.T
    )

    _scatter_pos, _gather = _sc_kernels()
    pos = _scatter_pos(item3d, pval4d)
    ue, ice, ite, qflat = _gather(
        user3d, item3d, item16_3d, pos.reshape(NUM_ITEMS * PR),
        users_rm, ic_rm, it_rm
    )

    lab3 = label.reshape(N_BLK, 1, CROWS)
    w3 = jnp.broadcast_to(
        mask.astype(jnp.float32)[:, None], (B, L)
    ).reshape(N_BLK, 1, CROWS)
    q3 = qflat.reshape(N_BLK, 1, CROWS)

    sums = _compute(ue, ice, ite, lab3, w3, q3)
    s_bce_c, s_bce_t, s_dc, s_dt, s_w, s_sw, s_win = (o[0, 0] for o in sums)

    seq_len = jnp.float32(L)
    cnt_t = s_w / seq_len
    cnt_c = jnp.float32(B) - cnt_t
    control_loss = s_bce_c / (cnt_c * seq_len)
    treatment_loss = s_bce_t / (cnt_t * seq_len)
    control_distance = s_dc / (cnt_c * seq_len)
    treatment_distance = s_dt / (cnt_t * seq_len)
    discrepancy_loss = s_sw / (s_win * jnp.float32(EMB))
    return (control_loss, treatment_loss, discrepancy_loss,
            control_distance, treatment_distance)


# speedup vs baseline: 1.7166x; 1.7166x over previous
import functools

import jax
import jax.numpy as jnp
from jax import lax
from jax.experimental import pallas as pl
from jax.experimental.pallas import tpu as pltpu
from jax.experimental.pallas import tpu_sc as plsc

NUM_USERS = 1000000
NUM_ITEMS = 1000000
EMB = 32
B, L = 4096, 50
N = B * L
CH = 128
NC, NS = 2, 16
NW = NC * NS
CPT = N // (NW * CH)
PR = 16
KG = 10
CROWS = 1024
N_BLK = N // CROWS
TBW = 8192


def _scatter_pos_body(item_hbm, pval_hbm, pos_hbm, idx_v, src_v, sem):
    wid = lax.axis_index("s") * NC + lax.axis_index("c")
    pltpu.sync_copy(item_hbm.at[wid], idx_v)

    def group(g, carry):
        pltpu.sync_copy(pval_hbm.at[wid, pl.ds(g * KG, KG)], src_v)
        for j in range(KG):
            pltpu.async_copy(
                src_v.at[j], pos_hbm.at[idx_v.at[g * KG + j]], sem
            )
        for j in range(KG):
            pltpu.make_async_copy(
                src_v.at[j], pos_hbm.at[idx_v.at[g * KG + j]], sem
            ).wait()
        return carry

    lax.fori_loop(0, CPT // KG, group, 0)


def _gather_body(user_hbm, item_hbm, item16_hbm, pos_hbm, users_hbm, ic_hbm,
                 it_hbm, ue_hbm, ice_hbm, ite_hbm, q_hbm,
                 uidx_v, iidx_v, i16_v, ubuf, cbuf, tbuf, qacc_v, gsem):
    wid = lax.axis_index("s") * NC + lax.axis_index("c")
    base = wid * CPT
    pltpu.sync_copy(user_hbm.at[wid], uidx_v)
    pltpu.sync_copy(item_hbm.at[wid], iidx_v)
    pltpu.sync_copy(item16_hbm.at[wid], i16_v)

    def chunk(j, carry):
        cu = pltpu.async_copy(users_hbm.at[uidx_v.at[j]], ubuf, gsem)
        cc = pltpu.async_copy(ic_hbm.at[iidx_v.at[j]], cbuf, gsem)
        ct = pltpu.async_copy(it_hbm.at[iidx_v.at[j]], tbuf, gsem)
        cq = pltpu.async_copy(
            pos_hbm.at[i16_v.at[j]], qacc_v.at[pl.ds(j * CH, CH)], gsem
        )
        cu.wait()
        cc.wait()
        ct.wait()
        cq.wait()
        p0 = (base + j) * CH
        b = p0 // CROWS
        rr0 = p0 % CROWS
        sl = pl.ds(0, EMB)
        pltpu.sync_copy(ubuf.at[:, sl], ue_hbm.at[b, pl.ds(rr0, CH), sl])
        pltpu.sync_copy(cbuf.at[:, sl], ice_hbm.at[b, pl.ds(rr0, CH), sl])
        pltpu.sync_copy(tbuf.at[:, sl], ite_hbm.at[b, pl.ds(rr0, CH), sl])
        return carry

    lax.fori_loop(0, CPT, chunk, 0)
    pltpu.sync_copy(qacc_v, q_hbm.at[pl.ds(wid * CPT * CH, CPT * CH)])


@functools.cache
def _sc_kernels():
    mesh = plsc.VectorSubcoreMesh(
        core_axis_name="c", subcore_axis_name="s", num_cores=NC, num_subcores=NS
    )
    params = pltpu.CompilerParams(use_tc_tiling_on_sc=False)
    scatter_pos = pl.kernel(
        _scatter_pos_body,
        out_type=jax.ShapeDtypeStruct((NUM_ITEMS, PR), jnp.int32),
        mesh=mesh,
        scratch_types=[
            pltpu.VMEM((CPT, CH), jnp.int32),
            pltpu.VMEM((KG, CH, PR), jnp.int32),
            pltpu.SemaphoreType.DMA,
        ],
        name="sc_scatter_pos",
        compiler_params=params,
    )
    gather = pl.kernel(
        _gather_body,
        out_type=(
            jax.ShapeDtypeStruct((N_BLK, CROWS, 128), jnp.float32),
            jax.ShapeDtypeStruct((N_BLK, CROWS, 128), jnp.float32),
            jax.ShapeDtypeStruct((N_BLK, CROWS, 128), jnp.float32),
            jax.ShapeDtypeStruct((N,), jnp.int32),
        ),
        mesh=mesh,
        scratch_types=[
            pltpu.VMEM((CPT, CH), jnp.int32),
            pltpu.VMEM((CPT, CH), jnp.int32),
            pltpu.VMEM((CPT, CH), jnp.int32),
            pltpu.VMEM((CH, 128), jnp.float32),
            pltpu.VMEM((CH, 128), jnp.float32),
            pltpu.VMEM((CH, 128), jnp.float32),
            pltpu.VMEM((CPT * CH,), jnp.int32),
            pltpu.SemaphoreType.DMA,
        ],
        name="sc_gather",
        compiler_params=params,
    )
    return scatter_pos, gather


def _transpose_body(ut_ref, ct_ref, tt_ref, u_out, c_out, t_out):
    u_out[:, pl.ds(0, EMB)] = ut_ref[...].T
    c_out[:, pl.ds(0, EMB)] = ct_ref[...].T
    t_out[:, pl.ds(0, EMB)] = tt_ref[...].T


_N_TBLK = -(-NUM_ITEMS // TBW)
_transpose = pl.pallas_call(
    _transpose_body,
    grid=(_N_TBLK,),
    in_specs=[pl.BlockSpec((EMB, TBW), lambda i: (0, i))] * 3,
    out_specs=[pl.BlockSpec((TBW, 128), lambda i: (i, 0))] * 3,
    out_shape=[jax.ShapeDtypeStruct((NUM_ITEMS, 128), jnp.float32)] * 3,
)


def _compute_body(ue_ref, ice_ref, ite_ref, lab_ref, w_ref, q_ref, *outs):
    i = pl.program_id(0)
    u = ue_ref[0]
    c = ice_ref[0]
    t = ite_ref[0]
    lab = lab_ref[0]
    w = w_ref[0]
    q = q_ref[0]

    valid = (
        lax.broadcasted_iota(jnp.int32, (CROWS, 128), 1) < EMB
    )
    ones = jnp.ones((128, 1), jnp.float32)

    def rowsum_t(x):
        xz = jnp.where(valid, x, 0.0)
        return lax.dot_general(ones, xz, (((0,), (1,)), ((), ())))

    sc = rowsum_t(u * c)
    st = rowsum_t(u * t)
    d = c - t
    s = rowsum_t(d * d)

    pidx = i * CROWS + lax.broadcasted_iota(jnp.int32, (1, CROWS), 1)
    winf = (q == pidx).astype(jnp.float32)
    nw = 1.0 - w

    def bce(x):
        return jnp.maximum(x, 0.0) - x * lab + jnp.log1p(jnp.exp(-jnp.abs(x)))

    sig = lambda x: 1.0 / (1.0 + jnp.exp(-x))
    sums = (
        jnp.sum(bce(sc) * nw),
        jnp.sum(bce(st) * w),
        jnp.sum(jnp.abs(sig(sc) - lab) * nw),
        jnp.sum(jnp.abs(sig(st) - lab) * w),
        jnp.sum(w),
        jnp.sum(s * winf),
        jnp.sum(winf),
    )
    for o_ref, val in zip(outs, sums):
        @pl.when(i == 0)
        def _init(o_ref=o_ref):
            o_ref[...] = jnp.zeros_like(o_ref)

        o_ref[...] += val


_N_SUMS = 7
_compute = pl.pallas_call(
    _compute_body,
    grid=(N_BLK,),
    in_specs=[
        pl.BlockSpec((1, CROWS, 128), lambda i: (i, 0, 0)),
        pl.BlockSpec((1, CROWS, 128), lambda i: (i, 0, 0)),
        pl.BlockSpec((1, CROWS, 128), lambda i: (i, 0, 0)),
        pl.BlockSpec((1, 1, CROWS), lambda i: (i, 0, 0)),
        pl.BlockSpec((1, 1, CROWS), lambda i: (i, 0, 0)),
        pl.BlockSpec((1, 1, CROWS), lambda i: (i, 0, 0)),
    ],
    out_specs=[pl.BlockSpec((1, 128), lambda i: (0, 0))] * _N_SUMS,
    out_shape=[jax.ShapeDtypeStruct((1, 128), jnp.float32)] * _N_SUMS,
)


def kernel(user, item, label, mask, users, items_control, items_treatment):
    user3d = user.reshape(NW, CPT, CH)
    item3d = item.reshape(NW, CPT, CH)
    pval4d = jnp.broadcast_to(
        jnp.arange(N, dtype=jnp.int32)[:, None], (N, PR)
    ).reshape(NW, CPT, CH, PR)
    item16_3d = (item * PR).reshape(NW, CPT, CH)

    users_rm, ic_rm, it_rm = _transpose(
        users.T, items_control.T, items_treatment.T
    )

    _scatter_pos, _gather = _sc_kernels()
    pos = _scatter_pos(item3d, pval4d)
    ue, ice, ite, qflat = _gather(
        user3d, item3d, item16_3d, pos.reshape(NUM_ITEMS * PR),
        users_rm, ic_rm, it_rm
    )

    lab3 = label.reshape(N_BLK, 1, CROWS)
    w3 = jnp.broadcast_to(
        mask.astype(jnp.float32)[:, None], (B, L)
    ).reshape(N_BLK, 1, CROWS)
    q3 = qflat.reshape(N_BLK, 1, CROWS)

    sums = _compute(ue, ice, ite, lab3, w3, q3)
    s_bce_c, s_bce_t, s_dc, s_dt, s_w, s_sw, s_win = (o[0, 0] for o in sums)

    seq_len = jnp.float32(L)
    cnt_t = s_w / seq_len
    cnt_c = jnp.float32(B) - cnt_t
    control_loss = s_bce_c / (cnt_c * seq_len)
    treatment_loss = s_bce_t / (cnt_t * seq_len)
    control_distance = s_dc / (cnt_c * seq_len)
    treatment_distance = s_dt / (cnt_t * seq_len)
    discrepancy_loss = s_sw / (s_win * jnp.float32(EMB))
    return (control_loss, treatment_loss, discrepancy_loss,
            control_distance, treatment_distance)


# speedup vs baseline: 1.8930x; 1.1028x over previous
import functools

import jax
import jax.numpy as jnp
from jax import lax
from jax.experimental import pallas as pl
from jax.experimental.pallas import tpu as pltpu
from jax.experimental.pallas import tpu_sc as plsc

NUM_USERS = 1000000
NUM_ITEMS = 1000000
EMB = 32
B, L = 4096, 50
N = B * L
CH = 128
NC, NS = 2, 16
NW = NC * NS
CPT = N // (NW * CH)
PR = 16
KG = 10
CROWS = 8192
N_BLK = N // CROWS
NSUB = CROWS // 1024
TBW = 8192


def _scatter_pos_body(item_hbm, pval_hbm, pos_hbm, idx_v, src_v, sem):
    wid = lax.axis_index("s") * NC + lax.axis_index("c")
    pltpu.sync_copy(item_hbm.at[wid], idx_v)

    def group(g, carry):
        pltpu.sync_copy(pval_hbm.at[wid, pl.ds(g * KG, KG)], src_v)
        for j in range(KG):
            pltpu.async_copy(
                src_v.at[j], pos_hbm.at[idx_v.at[g * KG + j]], sem
            )
        for j in range(KG):
            pltpu.make_async_copy(
                src_v.at[j], pos_hbm.at[idx_v.at[g * KG + j]], sem
            ).wait()
        return carry

    lax.fori_loop(0, CPT // KG, group, 0)


def _gather_body(user_hbm, item_hbm, item16_hbm, pos_hbm, users_hbm, ic_hbm,
                 it_hbm, ue_hbm, ice_hbm, ite_hbm, q_hbm,
                 uidx_v, iidx_v, i16_v, ubuf, cbuf, tbuf, qacc_v, gsem):
    wid = lax.axis_index("s") * NC + lax.axis_index("c")
    base = wid * CPT
    pltpu.sync_copy(user_hbm.at[wid], uidx_v)
    pltpu.sync_copy(item_hbm.at[wid], iidx_v)
    pltpu.sync_copy(item16_hbm.at[wid], i16_v)

    def chunk(j, carry):
        cu = pltpu.async_copy(users_hbm.at[uidx_v.at[j]], ubuf, gsem)
        cc = pltpu.async_copy(ic_hbm.at[iidx_v.at[j]], cbuf, gsem)
        ct = pltpu.async_copy(it_hbm.at[iidx_v.at[j]], tbuf, gsem)
        cq = pltpu.async_copy(
            pos_hbm.at[i16_v.at[j]], qacc_v.at[pl.ds(j * CH, CH)], gsem
        )
        cu.wait()
        cc.wait()
        ct.wait()
        cq.wait()
        p0 = (base + j) * CH
        b = p0 // CROWS
        rr0 = p0 % CROWS
        sl = pl.ds(0, EMB)
        pltpu.sync_copy(ubuf.at[:, sl], ue_hbm.at[b, pl.ds(rr0, CH), sl])
        pltpu.sync_copy(cbuf.at[:, sl], ice_hbm.at[b, pl.ds(rr0, CH), sl])
        pltpu.sync_copy(tbuf.at[:, sl], ite_hbm.at[b, pl.ds(rr0, CH), sl])
        return carry

    lax.fori_loop(0, CPT, chunk, 0)
    pltpu.sync_copy(qacc_v, q_hbm.at[pl.ds(wid * CPT * CH, CPT * CH)])


@functools.cache
def _sc_kernels():
    mesh = plsc.VectorSubcoreMesh(
        core_axis_name="c", subcore_axis_name="s", num_cores=NC, num_subcores=NS
    )
    params = pltpu.CompilerParams(use_tc_tiling_on_sc=False)
    scatter_pos = pl.kernel(
        _scatter_pos_body,
        out_type=jax.ShapeDtypeStruct((NUM_ITEMS, PR), jnp.int32),
        mesh=mesh,
        scratch_types=[
            pltpu.VMEM((CPT, CH), jnp.int32),
            pltpu.VMEM((KG, CH, PR), jnp.int32),
            pltpu.SemaphoreType.DMA,
        ],
        name="sc_scatter_pos",
        compiler_params=params,
    )
    gather = pl.kernel(
        _gather_body,
        out_type=(
            jax.ShapeDtypeStruct((N_BLK, CROWS, 128), jnp.float32),
            jax.ShapeDtypeStruct((N_BLK, CROWS, 128), jnp.float32),
            jax.ShapeDtypeStruct((N_BLK, CROWS, 128), jnp.float32),
            jax.ShapeDtypeStruct((N,), jnp.int32),
        ),
        mesh=mesh,
        scratch_types=[
            pltpu.VMEM((CPT, CH), jnp.int32),
            pltpu.VMEM((CPT, CH), jnp.int32),
            pltpu.VMEM((CPT, CH), jnp.int32),
            pltpu.VMEM((CH, 128), jnp.float32),
            pltpu.VMEM((CH, 128), jnp.float32),
            pltpu.VMEM((CH, 128), jnp.float32),
            pltpu.VMEM((CPT * CH,), jnp.int32),
            pltpu.SemaphoreType.DMA,
        ],
        name="sc_gather",
        compiler_params=params,
    )
    return scatter_pos, gather


_TSL = 4


def _transpose_body(ut_ref, ct_ref, tt_ref, u_out, c_out, t_out):
    sl = TBW // _TSL
    for s in range(_TSL):
        cols = pl.ds(s * sl, sl)
        rows = pl.ds(s * sl, sl)
        u_out[rows, pl.ds(0, EMB)] = ut_ref[:, cols].T
        c_out[rows, pl.ds(0, EMB)] = ct_ref[:, cols].T
        t_out[rows, pl.ds(0, EMB)] = tt_ref[:, cols].T


_N_TBLK = -(-NUM_ITEMS // TBW)
_transpose = pl.pallas_call(
    _transpose_body,
    grid=(_N_TBLK,),
    in_specs=[pl.BlockSpec((EMB, TBW), lambda i: (0, i))] * 3,
    out_specs=[pl.BlockSpec((TBW, 128), lambda i: (i, 0))] * 3,
    out_shape=[jax.ShapeDtypeStruct((NUM_ITEMS, 128), jnp.float32)] * 3,
)


def _compute_body(ue_ref, ice_ref, ite_ref, lab_ref, w_ref, q_ref, *outs):
    i = pl.program_id(0)
    u = ue_ref[0]
    c = ice_ref[0]
    t = ite_ref[0]
    lab = lab_ref[0]
    w = w_ref[0]
    q = q_ref[0]

    valid = (
        lax.broadcasted_iota(jnp.int32, (CROWS, 128), 1) < EMB
    )
    ones = jnp.ones((128, 1), jnp.float32)

    def rowsum_t(x):
        xz = jnp.where(valid, x, 0.0)
        rows = [
            lax.dot_general(
                ones,
                lax.slice(xz, (s * 1024, 0), ((s + 1) * 1024, 128)),
                (((0,), (1,)), ((), ())),
            )
            for s in range(NSUB)
        ]
        return jnp.concatenate(rows, axis=0)

    sc = rowsum_t(u * c)
    st = rowsum_t(u * t)
    d = c - t
    s = rowsum_t(d * d)

    pidx = (
        i * CROWS
        + lax.broadcasted_iota(jnp.int32, (NSUB, 1024), 0) * 1024
        + lax.broadcasted_iota(jnp.int32, (NSUB, 1024), 1)
    )
    winf = (q == pidx).astype(jnp.float32)
    nw = 1.0 - w

    def bce(x):
        return jnp.maximum(x, 0.0) - x * lab + jnp.log1p(jnp.exp(-jnp.abs(x)))

    sig = lambda x: 1.0 / (1.0 + jnp.exp(-x))
    sums = (
        jnp.sum(bce(sc) * nw),
        jnp.sum(bce(st) * w),
        jnp.sum(jnp.abs(sig(sc) - lab) * nw),
        jnp.sum(jnp.abs(sig(st) - lab) * w),
        jnp.sum(w),
        jnp.sum(s * winf),
        jnp.sum(winf),
    )
    for o_ref, val in zip(outs, sums):
        @pl.when(i == 0)
        def _init(o_ref=o_ref):
            o_ref[...] = jnp.zeros_like(o_ref)

        o_ref[...] += val


_N_SUMS = 7
_compute = pl.pallas_call(
    _compute_body,
    grid=(N_BLK,),
    in_specs=[
        pl.BlockSpec((1, CROWS, 128), lambda i: (i, 0, 0)),
        pl.BlockSpec((1, CROWS, 128), lambda i: (i, 0, 0)),
        pl.BlockSpec((1, CROWS, 128), lambda i: (i, 0, 0)),
        pl.BlockSpec((1, NSUB, 1024), lambda i: (i, 0, 0)),
        pl.BlockSpec((1, NSUB, 1024), lambda i: (i, 0, 0)),
        pl.BlockSpec((1, NSUB, 1024), lambda i: (i, 0, 0)),
    ],
    out_specs=[pl.BlockSpec((1, 128), lambda i: (0, 0))] * _N_SUMS,
    out_shape=[jax.ShapeDtypeStruct((1, 128), jnp.float32)] * _N_SUMS,
)


def kernel(user, item, label, mask, users, items_control, items_treatment):
    user3d = user.reshape(NW, CPT, CH)
    item3d = item.reshape(NW, CPT, CH)
    pval4d = jnp.broadcast_to(
        jnp.arange(N, dtype=jnp.int32)[:, None], (N, PR)
    ).reshape(NW, CPT, CH, PR)
    item16_3d = (item * PR).reshape(NW, CPT, CH)

    users_rm, ic_rm, it_rm = _transpose(
        users.T, items_control.T, items_treatment.T
    )

    _scatter_pos, _gather = _sc_kernels()
    pos = _scatter_pos(item3d, pval4d)
    ue, ice, ite, qflat = _gather(
        user3d, item3d, item16_3d, pos.reshape(NUM_ITEMS * PR),
        users_rm, ic_rm, it_rm
    )

    lab3 = label.reshape(N_BLK, NSUB, 1024)
    w3 = jnp.broadcast_to(
        mask.astype(jnp.float32)[:, None], (B, L)
    ).reshape(N_BLK, NSUB, 1024)
    q3 = qflat.reshape(N_BLK, NSUB, 1024)

    sums = _compute(ue, ice, ite, lab3, w3, q3)
    s_bce_c, s_bce_t, s_dc, s_dt, s_w, s_sw, s_win = (o[0, 0] for o in sums)

    seq_len = jnp.float32(L)
    cnt_t = s_w / seq_len
    cnt_c = jnp.float32(B) - cnt_t
    control_loss = s_bce_c / (cnt_c * seq_len)
    treatment_loss = s_bce_t / (cnt_t * seq_len)
    control_distance = s_dc / (cnt_c * seq_len)
    treatment_distance = s_dt / (cnt_t * seq_len)
    discrepancy_loss = s_sw / (s_win * jnp.float32(EMB))
    return (control_loss, treatment_loss, discrepancy_loss,
            control_distance, treatment_distance)


# speedup vs baseline: 1.9114x; 1.0097x over previous
import functools

import jax
import jax.numpy as jnp
from jax import lax
from jax.experimental import pallas as pl
from jax.experimental.pallas import tpu as pltpu
from jax.experimental.pallas import tpu_sc as plsc

NUM_USERS = 1000000
NUM_ITEMS = 1000000
EMB = 32
B, L = 4096, 50
N = B * L
CH = 128
NC, NS = 2, 16
NW = NC * NS
CPT = N // (NW * CH)
PR = 16
KG = 10
CROWS = 8192
N_BLK = N // CROWS
NSUB = CROWS // 1024
TBW = 16384


def _scatter_pos_body(item_hbm, pval_hbm, pos_hbm, idx_v, src_v, sem):
    wid = lax.axis_index("s") * NC + lax.axis_index("c")
    pltpu.sync_copy(item_hbm.at[wid], idx_v)

    def group(g, carry):
        pltpu.sync_copy(pval_hbm.at[wid, pl.ds(g * KG, KG)], src_v)
        for j in range(KG):
            pltpu.async_copy(
                src_v.at[j], pos_hbm.at[idx_v.at[g * KG + j]], sem
            )
        for j in range(KG):
            pltpu.make_async_copy(
                src_v.at[j], pos_hbm.at[idx_v.at[g * KG + j]], sem
            ).wait()
        return carry

    lax.fori_loop(0, CPT // KG, group, 0)


def _gather_body(user_hbm, item_hbm, item16_hbm, pos_hbm, users_hbm, ic_hbm,
                 it_hbm, ue_hbm, ice_hbm, ite_hbm, q_hbm,
                 uidx_v, iidx_v, i16_v, ubuf, cbuf, tbuf, qacc_v, gsem):
    wid = lax.axis_index("s") * NC + lax.axis_index("c")
    base = wid * CPT
    pltpu.sync_copy(user_hbm.at[wid], uidx_v)
    pltpu.sync_copy(item_hbm.at[wid], iidx_v)
    pltpu.sync_copy(item16_hbm.at[wid], i16_v)

    def chunk(j, carry):
        cu = pltpu.async_copy(users_hbm.at[uidx_v.at[j]], ubuf, gsem)
        cc = pltpu.async_copy(ic_hbm.at[iidx_v.at[j]], cbuf, gsem)
        ct = pltpu.async_copy(it_hbm.at[iidx_v.at[j]], tbuf, gsem)
        cq = pltpu.async_copy(
            pos_hbm.at[i16_v.at[j]], qacc_v.at[pl.ds(j * CH, CH)], gsem
        )
        cu.wait()
        cc.wait()
        ct.wait()
        cq.wait()
        p0 = (base + j) * CH
        b = p0 // CROWS
        rr0 = p0 % CROWS
        sl = pl.ds(0, EMB)
        pltpu.sync_copy(ubuf.at[:, sl], ue_hbm.at[b, pl.ds(rr0, CH), sl])
        pltpu.sync_copy(cbuf.at[:, sl], ice_hbm.at[b, pl.ds(rr0, CH), sl])
        pltpu.sync_copy(tbuf.at[:, sl], ite_hbm.at[b, pl.ds(rr0, CH), sl])
        return carry

    lax.fori_loop(0, CPT, chunk, 0)
    pltpu.sync_copy(qacc_v, q_hbm.at[pl.ds(wid * CPT * CH, CPT * CH)])


@functools.cache
def _sc_kernels():
    mesh = plsc.VectorSubcoreMesh(
        core_axis_name="c", subcore_axis_name="s", num_cores=NC, num_subcores=NS
    )
    params = pltpu.CompilerParams(use_tc_tiling_on_sc=False)
    scatter_pos = pl.kernel(
        _scatter_pos_body,
        out_type=jax.ShapeDtypeStruct((NUM_ITEMS, PR), jnp.int32),
        mesh=mesh,
        scratch_types=[
            pltpu.VMEM((CPT, CH), jnp.int32),
            pltpu.VMEM((KG, CH, PR), jnp.int32),
            pltpu.SemaphoreType.DMA,
        ],
        name="sc_scatter_pos",
        compiler_params=params,
    )
    gather = pl.kernel(
        _gather_body,
        out_type=(
            jax.ShapeDtypeStruct((N_BLK, CROWS, 128), jnp.float32),
            jax.ShapeDtypeStruct((N_BLK, CROWS, 128), jnp.float32),
            jax.ShapeDtypeStruct((N_BLK, CROWS, 128), jnp.float32),
            jax.ShapeDtypeStruct((N,), jnp.int32),
        ),
        mesh=mesh,
        scratch_types=[
            pltpu.VMEM((CPT, CH), jnp.int32),
            pltpu.VMEM((CPT, CH), jnp.int32),
            pltpu.VMEM((CPT, CH), jnp.int32),
            pltpu.VMEM((CH, 128), jnp.float32),
            pltpu.VMEM((CH, 128), jnp.float32),
            pltpu.VMEM((CH, 128), jnp.float32),
            pltpu.VMEM((CPT * CH,), jnp.int32),
            pltpu.SemaphoreType.DMA,
        ],
        name="sc_gather",
        compiler_params=params,
    )
    return scatter_pos, gather


_TSL = 4


def _transpose_body(ut_ref, ct_ref, tt_ref, u_out, c_out, t_out):
    sl = TBW // _TSL
    for s in range(_TSL):
        cols = pl.ds(s * sl, sl)
        rows = pl.ds(s * sl, sl)
        u_out[rows, pl.ds(0, EMB)] = ut_ref[:, cols].T
        c_out[rows, pl.ds(0, EMB)] = ct_ref[:, cols].T
        t_out[rows, pl.ds(0, EMB)] = tt_ref[:, cols].T


_N_TBLK = -(-NUM_ITEMS // TBW)
_transpose = pl.pallas_call(
    _transpose_body,
    grid=(_N_TBLK,),
    in_specs=[pl.BlockSpec((EMB, TBW), lambda i: (0, i))] * 3,
    out_specs=[pl.BlockSpec((TBW, 128), lambda i: (i, 0))] * 3,
    out_shape=[jax.ShapeDtypeStruct((NUM_ITEMS, 128), jnp.float32)] * 3,
    compiler_params=pltpu.CompilerParams(vmem_limit_bytes=100 * 1024 * 1024),
)


def _compute_body(ue_ref, ice_ref, ite_ref, lab_ref, w_ref, q_ref, *outs):
    i = pl.program_id(0)
    u = ue_ref[0]
    c = ice_ref[0]
    t = ite_ref[0]
    lab = lab_ref[0]
    w = w_ref[0]
    q = q_ref[0]

    valid = (
        lax.broadcasted_iota(jnp.int32, (CROWS, 128), 1) < EMB
    )
    ones = jnp.ones((128, 1), jnp.float32)

    def rowsum_t(x):
        xz = jnp.where(valid, x, 0.0)
        rows = [
            lax.dot_general(
                ones,
                lax.slice(xz, (s * 1024, 0), ((s + 1) * 1024, 128)),
                (((0,), (1,)), ((), ())),
            )
            for s in range(NSUB)
        ]
        return jnp.concatenate(rows, axis=0)

    sc = rowsum_t(u * c)
    st = rowsum_t(u * t)
    d = c - t
    s = rowsum_t(d * d)

    pidx = (
        i * CROWS
        + lax.broadcasted_iota(jnp.int32, (NSUB, 1024), 0) * 1024
        + lax.broadcasted_iota(jnp.int32, (NSUB, 1024), 1)
    )
    winf = (q == pidx).astype(jnp.float32)
    nw = 1.0 - w

    def bce(x):
        return jnp.maximum(x, 0.0) - x * lab + jnp.log1p(jnp.exp(-jnp.abs(x)))

    sig = lambda x: 1.0 / (1.0 + jnp.exp(-x))
    sums = (
        jnp.sum(bce(sc) * nw),
        jnp.sum(bce(st) * w),
        jnp.sum(jnp.abs(sig(sc) - lab) * nw),
        jnp.sum(jnp.abs(sig(st) - lab) * w),
        jnp.sum(w),
        jnp.sum(s * winf),
        jnp.sum(winf),
    )
    for o_ref, val in zip(outs, sums):
        @pl.when(i == 0)
        def _init(o_ref=o_ref):
            o_ref[...] = jnp.zeros_like(o_ref)

        o_ref[...] += val


_N_SUMS = 7
_compute = pl.pallas_call(
    _compute_body,
    grid=(N_BLK,),
    in_specs=[
        pl.BlockSpec((1, CROWS, 128), lambda i: (i, 0, 0)),
        pl.BlockSpec((1, CROWS, 128), lambda i: (i, 0, 0)),
        pl.BlockSpec((1, CROWS, 128), lambda i: (i, 0, 0)),
        pl.BlockSpec((1, NSUB, 1024), lambda i: (i, 0, 0)),
        pl.BlockSpec((1, NSUB, 1024), lambda i: (i, 0, 0)),
        pl.BlockSpec((1, NSUB, 1024), lambda i: (i, 0, 0)),
    ],
    out_specs=[pl.BlockSpec((1, 128), lambda i: (0, 0))] * _N_SUMS,
    out_shape=[jax.ShapeDtypeStruct((1, 128), jnp.float32)] * _N_SUMS,
)


def kernel(user, item, label, mask, users, items_control, items_treatment):
    user3d = user.reshape(NW, CPT, CH)
    item3d = item.reshape(NW, CPT, CH)
    pval4d = jnp.broadcast_to(
        jnp.arange(N, dtype=jnp.int32)[:, None], (N, PR)
    ).reshape(NW, CPT, CH, PR)
    item16_3d = (item * PR).reshape(NW, CPT, CH)

    users_rm, ic_rm, it_rm = _transpose(
        users.T, items_control.T, items_treatment.T
    )

    _scatter_pos, _gather = _sc_kernels()
    pos = _scatter_pos(item3d, pval4d)
    ue, ice, ite, qflat = _gather(
        user3d, item3d, item16_3d, pos.reshape(NUM_ITEMS * PR),
        users_rm, ic_rm, it_rm
    )

    lab3 = label.reshape(N_BLK, NSUB, 1024)
    w3 = jnp.broadcast_to(
        mask.astype(jnp.float32)[:, None], (B, L)
    ).reshape(N_BLK, NSUB, 1024)
    q3 = qflat.reshape(N_BLK, NSUB, 1024)

    sums = _compute(ue, ice, ite, lab3, w3, q3)
    s_bce_c, s_bce_t, s_dc, s_dt, s_w, s_sw, s_win = (o[0, 0] for o in sums)

    seq_len = jnp.float32(L)
    cnt_t = s_w / seq_len
    cnt_c = jnp.float32(B) - cnt_t
    control_loss = s_bce_c / (cnt_c * seq_len)
    treatment_loss = s_bce_t / (cnt_t * seq_len)
    control_distance = s_dc / (cnt_c * seq_len)
    treatment_distance = s_dt / (cnt_t * seq_len)
    discrepancy_loss = s_sw / (s_win * jnp.float32(EMB))
    return (control_loss, treatment_loss, discrepancy_loss,
            control_distance, treatment_distance)


# speedup vs baseline: 1.9126x; 1.0006x over previous
import functools

import jax
import jax.numpy as jnp
from jax import lax
from jax.experimental import pallas as pl
from jax.experimental.pallas import tpu as pltpu
from jax.experimental.pallas import tpu_sc as plsc

NUM_USERS = 1000000
NUM_ITEMS = 1000000
EMB = 32
B, L = 4096, 50
N = B * L
CH = 128
NC, NS = 2, 16
NW = NC * NS
CPT = N // (NW * CH)
PR = 16
KG = 10
CROWS = 8192
N_BLK = N // CROWS
NSUB = CROWS // 1024
TBW = 16384


def _scatter_pos_body(item_hbm, pval_hbm, pos_hbm, idx_v, src_v, sem):
    wid = lax.axis_index("s") * NC + lax.axis_index("c")
    pltpu.sync_copy(item_hbm.at[wid], idx_v)

    def group(g, carry):
        pltpu.sync_copy(pval_hbm.at[wid, pl.ds(g * KG, KG)], src_v)
        for j in range(KG):
            pltpu.async_copy(
                src_v.at[j], pos_hbm.at[idx_v.at[g * KG + j]], sem
            )
        for j in range(KG):
            pltpu.make_async_copy(
                src_v.at[j], pos_hbm.at[idx_v.at[g * KG + j]], sem
            ).wait()
        return carry

    lax.fori_loop(0, CPT // KG, group, 0)


def _gather_one_body(idx_hbm, tab_hbm, out_hbm, idx_v, buf, gsem):
    wid = lax.axis_index("s") * NC + lax.axis_index("c")
    base = wid * CPT
    pltpu.sync_copy(idx_hbm.at[wid], idx_v)

    def chunk(j, carry):
        cg = pltpu.async_copy(tab_hbm.at[idx_v.at[j]], buf, gsem)
        cg.wait()
        p0 = (base + j) * CH
        sl = pl.ds(0, EMB)
        pltpu.sync_copy(
            buf.at[:, sl],
            out_hbm.at[p0 // CROWS, pl.ds(p0 % CROWS, CH), sl],
        )
        return carry

    lax.fori_loop(0, CPT, chunk, 0)


def _gather_it_body(item_hbm, item16_hbm, pos_hbm, it_hbm, ite_hbm, q_hbm,
                    iidx_v, i16_v, tbuf, qacc_v, gsem):
    wid = lax.axis_index("s") * NC + lax.axis_index("c")
    base = wid * CPT
    pltpu.sync_copy(item_hbm.at[wid], iidx_v)
    pltpu.sync_copy(item16_hbm.at[wid], i16_v)

    def chunk(j, carry):
        ct = pltpu.async_copy(it_hbm.at[iidx_v.at[j]], tbuf, gsem)
        cq = pltpu.async_copy(
            pos_hbm.at[i16_v.at[j]], qacc_v.at[pl.ds(j * CH, CH)], gsem
        )
        ct.wait()
        cq.wait()
        p0 = (base + j) * CH
        sl = pl.ds(0, EMB)
        pltpu.sync_copy(
            tbuf.at[:, sl],
            ite_hbm.at[p0 // CROWS, pl.ds(p0 % CROWS, CH), sl],
        )
        return carry

    lax.fori_loop(0, CPT, chunk, 0)
    pltpu.sync_copy(qacc_v, q_hbm.at[pl.ds(wid * CPT * CH, CPT * CH)])


@functools.cache
def _sc_kernels():
    mesh = plsc.VectorSubcoreMesh(
        core_axis_name="c", subcore_axis_name="s", num_cores=NC, num_subcores=NS
    )
    params = pltpu.CompilerParams(use_tc_tiling_on_sc=False)
    scatter_pos = pl.kernel(
        _scatter_pos_body,
        out_type=jax.ShapeDtypeStruct((NUM_ITEMS, PR), jnp.int32),
        mesh=mesh,
        scratch_types=[
            pltpu.VMEM((CPT, CH), jnp.int32),
            pltpu.VMEM((KG, CH, PR), jnp.int32),
            pltpu.SemaphoreType.DMA,
        ],
        name="sc_scatter_pos",
        compiler_params=params,
    )
    gather_one = pl.kernel(
        _gather_one_body,
        out_type=jax.ShapeDtypeStruct((N_BLK, CROWS, 128), jnp.float32),
        mesh=mesh,
        scratch_types=[
            pltpu.VMEM((CPT, CH), jnp.int32),
            pltpu.VMEM((CH, 128), jnp.float32),
            pltpu.SemaphoreType.DMA,
        ],
        name="sc_gather_one",
        compiler_params=params,
    )
    gather_it = pl.kernel(
        _gather_it_body,
        out_type=(
            jax.ShapeDtypeStruct((N_BLK, CROWS, 128), jnp.float32),
            jax.ShapeDtypeStruct((N,), jnp.int32),
        ),
        mesh=mesh,
        scratch_types=[
            pltpu.VMEM((CPT, CH), jnp.int32),
            pltpu.VMEM((CPT, CH), jnp.int32),
            pltpu.VMEM((CH, 128), jnp.float32),
            pltpu.VMEM((CPT * CH,), jnp.int32),
            pltpu.SemaphoreType.DMA,
        ],
        name="sc_gather_it",
        compiler_params=params,
    )
    return scatter_pos, gather_one, gather_it


_TSL = 4


def _transpose_body(t_ref, out_ref):
    sl = TBW // _TSL
    for s in range(_TSL):
        cols = pl.ds(s * sl, sl)
        rows = pl.ds(s * sl, sl)
        out_ref[rows, pl.ds(0, EMB)] = t_ref[:, cols].T


_N_TBLK = -(-NUM_ITEMS // TBW)
_transpose1 = pl.pallas_call(
    _transpose_body,
    grid=(_N_TBLK,),
    in_specs=[pl.BlockSpec((EMB, TBW), lambda i: (0, i))],
    out_specs=pl.BlockSpec((TBW, 128), lambda i: (i, 0)),
    out_shape=jax.ShapeDtypeStruct((NUM_ITEMS, 128), jnp.float32),
    compiler_params=pltpu.CompilerParams(vmem_limit_bytes=100 * 1024 * 1024),
)


def _compute_body(ue_ref, ice_ref, ite_ref, lab_ref, w_ref, q_ref, *outs):
    i = pl.program_id(0)
    u = ue_ref[0]
    c = ice_ref[0]
    t = ite_ref[0]
    lab = lab_ref[0]
    w = w_ref[0]
    q = q_ref[0]

    valid = (
        lax.broadcasted_iota(jnp.int32, (CROWS, 128), 1) < EMB
    )
    ones = jnp.ones((128, 1), jnp.float32)

    def rowsum_t(x):
        xz = jnp.where(valid, x, 0.0)
        rows = [
            lax.dot_general(
                ones,
                lax.slice(xz, (s * 1024, 0), ((s + 1) * 1024, 128)),
                (((0,), (1,)), ((), ())),
            )
            for s in range(NSUB)
        ]
        return jnp.concatenate(rows, axis=0)

    sc = rowsum_t(u * c)
    st = rowsum_t(u * t)
    d = c - t
    s = rowsum_t(d * d)

    pidx = (
        i * CROWS
        + lax.broadcasted_iota(jnp.int32, (NSUB, 1024), 0) * 1024
        + lax.broadcasted_iota(jnp.int32, (NSUB, 1024), 1)
    )
    winf = (q == pidx).astype(jnp.float32)
    nw = 1.0 - w

    def bce(x):
        return jnp.maximum(x, 0.0) - x * lab + jnp.log1p(jnp.exp(-jnp.abs(x)))

    sig = lambda x: 1.0 / (1.0 + jnp.exp(-x))
    sums = (
        jnp.sum(bce(sc) * nw),
        jnp.sum(bce(st) * w),
        jnp.sum(jnp.abs(sig(sc) - lab) * nw),
        jnp.sum(jnp.abs(sig(st) - lab) * w),
        jnp.sum(w),
        jnp.sum(s * winf),
        jnp.sum(winf),
    )
    for o_ref, val in zip(outs, sums):
        @pl.when(i == 0)
        def _init(o_ref=o_ref):
            o_ref[...] = jnp.zeros_like(o_ref)

        o_ref[...] += val


_N_SUMS = 7
_compute = pl.pallas_call(
    _compute_body,
    grid=(N_BLK,),
    in_specs=[
        pl.BlockSpec((1, CROWS, 128), lambda i: (i, 0, 0)),
        pl.BlockSpec((1, CROWS, 128), lambda i: (i, 0, 0)),
        pl.BlockSpec((1, CROWS, 128), lambda i: (i, 0, 0)),
        pl.BlockSpec((1, NSUB, 1024), lambda i: (i, 0, 0)),
        pl.BlockSpec((1, NSUB, 1024), lambda i: (i, 0, 0)),
        pl.BlockSpec((1, NSUB, 1024), lambda i: (i, 0, 0)),
    ],
    out_specs=[pl.BlockSpec((1, 128), lambda i: (0, 0))] * _N_SUMS,
    out_shape=[jax.ShapeDtypeStruct((1, 128), jnp.float32)] * _N_SUMS,
)


def kernel(user, item, label, mask, users, items_control, items_treatment):
    user3d = user.reshape(NW, CPT, CH)
    item3d = item.reshape(NW, CPT, CH)
    pval4d = jnp.broadcast_to(
        jnp.arange(N, dtype=jnp.int32)[:, None], (N, PR)
    ).reshape(NW, CPT, CH, PR)
    item16_3d = (item * PR).reshape(NW, CPT, CH)

    _scatter_pos, _gather_one, _gather_it = _sc_kernels()
    pos = _scatter_pos(item3d, pval4d)

    users_rm = _transpose1(users.T)
    ue = _gather_one(user3d, users_rm)
    ic_rm = _transpose1(items_control.T)
    ice = _gather_one(item3d, ic_rm)
    it_rm = _transpose1(items_treatment.T)
    ite, qflat = _gather_it(
        item3d, item16_3d, pos.reshape(NUM_ITEMS * PR), it_rm
    )

    lab3 = label.reshape(N_BLK, NSUB, 1024)
    w3 = jnp.broadcast_to(
        mask.astype(jnp.float32)[:, None], (B, L)
    ).reshape(N_BLK, NSUB, 1024)
    q3 = qflat.reshape(N_BLK, NSUB, 1024)

    sums = _compute(ue, ice, ite, lab3, w3, q3)
    s_bce_c, s_bce_t, s_dc, s_dt, s_w, s_sw, s_win = (o[0, 0] for o in sums)

    seq_len = jnp.float32(L)
    cnt_t = s_w / seq_len
    cnt_c = jnp.float32(B) - cnt_t
    control_loss = s_bce_c / (cnt_c * seq_len)
    treatment_loss = s_bce_t / (cnt_t * seq_len)
    control_distance = s_dc / (cnt_c * seq_len)
    treatment_distance = s_dt / (cnt_t * seq_len)
    discrepancy_loss = s_sw / (s_win * jnp.float32(EMB))
    return (control_loss, treatment_loss, discrepancy_loss,
            control_distance, treatment_distance)


# speedup vs baseline: 1.9345x; 1.0115x over previous
import functools

import jax
import jax.numpy as jnp
from jax import lax
from jax.experimental import pallas as pl
from jax.experimental.pallas import tpu as pltpu
from jax.experimental.pallas import tpu_sc as plsc

NUM_USERS = 1000000
NUM_ITEMS = 1000000
EMB = 32
B, L = 4096, 50
N = B * L
CH = 128
NC, NS = 2, 16
NW = NC * NS
CPT = N // (NW * CH)
PR = 16
KG = 10
CROWS = 8192
N_BLK = N // CROWS
NSUB = CROWS // 1024
TBW = 32768


def _scatter_pos_body(item_hbm, pval_hbm, pos_hbm, idx_v, src_v, sem):
    wid = lax.axis_index("s") * NC + lax.axis_index("c")
    pltpu.sync_copy(item_hbm.at[wid], idx_v)

    def group(g, carry):
        pltpu.sync_copy(pval_hbm.at[wid, pl.ds(g * KG, KG)], src_v)
        for j in range(KG):
            pltpu.async_copy(
                src_v.at[j], pos_hbm.at[idx_v.at[g * KG + j]], sem
            )
        for j in range(KG):
            pltpu.make_async_copy(
                src_v.at[j], pos_hbm.at[idx_v.at[g * KG + j]], sem
            ).wait()
        return carry

    lax.fori_loop(0, CPT // KG, group, 0)


def _gather_one_body(idx_hbm, tab_hbm, out_hbm, idx_v, buf, gsem):
    wid = lax.axis_index("s") * NC + lax.axis_index("c")
    base = wid * CPT
    pltpu.sync_copy(idx_hbm.at[wid], idx_v)

    def chunk(j, carry):
        cg = pltpu.async_copy(tab_hbm.at[idx_v.at[j]], buf, gsem)
        cg.wait()
        p0 = (base + j) * CH
        sl = pl.ds(0, EMB)
        pltpu.sync_copy(
            buf.at[:, sl],
            out_hbm.at[p0 // CROWS, pl.ds(p0 % CROWS, CH), sl],
        )
        return carry

    lax.fori_loop(0, CPT, chunk, 0)


def _gather_it_body(item_hbm, item16_hbm, pos_hbm, it_hbm, ite_hbm, q_hbm,
                    iidx_v, i16_v, tbuf, qacc_v, gsem):
    wid = lax.axis_index("s") * NC + lax.axis_index("c")
    base = wid * CPT
    pltpu.sync_copy(item_hbm.at[wid], iidx_v)
    pltpu.sync_copy(item16_hbm.at[wid], i16_v)

    def chunk(j, carry):
        ct = pltpu.async_copy(it_hbm.at[iidx_v.at[j]], tbuf, gsem)
        cq = pltpu.async_copy(
            pos_hbm.at[i16_v.at[j]], qacc_v.at[pl.ds(j * CH, CH)], gsem
        )
        ct.wait()
        cq.wait()
        p0 = (base + j) * CH
        sl = pl.ds(0, EMB)
        pltpu.sync_copy(
            tbuf.at[:, sl],
            ite_hbm.at[p0 // CROWS, pl.ds(p0 % CROWS, CH), sl],
        )
        return carry

    lax.fori_loop(0, CPT, chunk, 0)
    pltpu.sync_copy(qacc_v, q_hbm.at[pl.ds(wid * CPT * CH, CPT * CH)])


@functools.cache
def _sc_kernels():
    mesh = plsc.VectorSubcoreMesh(
        core_axis_name="c", subcore_axis_name="s", num_cores=NC, num_subcores=NS
    )
    params = pltpu.CompilerParams(use_tc_tiling_on_sc=False)
    scatter_pos = pl.kernel(
        _scatter_pos_body,
        out_type=jax.ShapeDtypeStruct((NUM_ITEMS, PR), jnp.int32),
        mesh=mesh,
        scratch_types=[
            pltpu.VMEM((CPT, CH), jnp.int32),
            pltpu.VMEM((KG, CH, PR), jnp.int32),
            pltpu.SemaphoreType.DMA,
        ],
        name="sc_scatter_pos",
        compiler_params=params,
    )
    gather_one = pl.kernel(
        _gather_one_body,
        out_type=jax.ShapeDtypeStruct((N_BLK, CROWS, 128), jnp.float32),
        mesh=mesh,
        scratch_types=[
            pltpu.VMEM((CPT, CH), jnp.int32),
            pltpu.VMEM((CH, 128), jnp.float32),
            pltpu.SemaphoreType.DMA,
        ],
        name="sc_gather_one",
        compiler_params=params,
    )
    gather_it = pl.kernel(
        _gather_it_body,
        out_type=(
            jax.ShapeDtypeStruct((N_BLK, CROWS, 128), jnp.float32),
            jax.ShapeDtypeStruct((N,), jnp.int32),
        ),
        mesh=mesh,
        scratch_types=[
            pltpu.VMEM((CPT, CH), jnp.int32),
            pltpu.VMEM((CPT, CH), jnp.int32),
            pltpu.VMEM((CH, 128), jnp.float32),
            pltpu.VMEM((CPT * CH,), jnp.int32),
            pltpu.SemaphoreType.DMA,
        ],
        name="sc_gather_it",
        compiler_params=params,
    )
    return scatter_pos, gather_one, gather_it


_TSL = 4


def _transpose_body(t_ref, out_ref):
    sl = TBW // _TSL
    for s in range(_TSL):
        cols = pl.ds(s * sl, sl)
        rows = pl.ds(s * sl, sl)
        out_ref[rows, pl.ds(0, EMB)] = t_ref[:, cols].T


_N_TBLK = -(-NUM_ITEMS // TBW)
_transpose1 = pl.pallas_call(
    _transpose_body,
    grid=(_N_TBLK,),
    in_specs=[pl.BlockSpec((EMB, TBW), lambda i: (0, i))],
    out_specs=pl.BlockSpec((TBW, 128), lambda i: (i, 0)),
    out_shape=jax.ShapeDtypeStruct((NUM_ITEMS, 128), jnp.float32),
    compiler_params=pltpu.CompilerParams(vmem_limit_bytes=100 * 1024 * 1024),
)


def _compute_body(ue_ref, ice_ref, ite_ref, lab_ref, w_ref, q_ref, *outs):
    i = pl.program_id(0)
    u = ue_ref[0]
    c = ice_ref[0]
    t = ite_ref[0]
    lab = lab_ref[0]
    w = w_ref[0]
    q = q_ref[0]

    valid = (
        lax.broadcasted_iota(jnp.int32, (CROWS, 128), 1) < EMB
    )
    ones = jnp.ones((128, 1), jnp.float32)

    def rowsum_t(x):
        xz = jnp.where(valid, x, 0.0)
        rows = [
            lax.dot_general(
                ones,
                lax.slice(xz, (s * 1024, 0), ((s + 1) * 1024, 128)),
                (((0,), (1,)), ((), ())),
            )
            for s in range(NSUB)
        ]
        return jnp.concatenate(rows, axis=0)

    sc = rowsum_t(u * c)
    st = rowsum_t(u * t)
    d = c - t
    s = rowsum_t(d * d)

    pidx = (
        i * CROWS
        + lax.broadcasted_iota(jnp.int32, (NSUB, 1024), 0) * 1024
        + lax.broadcasted_iota(jnp.int32, (NSUB, 1024), 1)
    )
    winf = (q == pidx).astype(jnp.float32)
    nw = 1.0 - w

    def bce(x):
        return jnp.maximum(x, 0.0) - x * lab + jnp.log1p(jnp.exp(-jnp.abs(x)))

    sig = lambda x: 1.0 / (1.0 + jnp.exp(-x))
    sums = (
        jnp.sum(bce(sc) * nw),
        jnp.sum(bce(st) * w),
        jnp.sum(jnp.abs(sig(sc) - lab) * nw),
        jnp.sum(jnp.abs(sig(st) - lab) * w),
        jnp.sum(w),
        jnp.sum(s * winf),
        jnp.sum(winf),
    )
    for o_ref, val in zip(outs, sums):
        @pl.when(i == 0)
        def _init(o_ref=o_ref):
            o_ref[...] = jnp.zeros_like(o_ref)

        o_ref[...] += val


_N_SUMS = 7
_compute = pl.pallas_call(
    _compute_body,
    grid=(N_BLK,),
    in_specs=[
        pl.BlockSpec((1, CROWS, 128), lambda i: (i, 0, 0)),
        pl.BlockSpec((1, CROWS, 128), lambda i: (i, 0, 0)),
        pl.BlockSpec((1, CROWS, 128), lambda i: (i, 0, 0)),
        pl.BlockSpec((1, NSUB, 1024), lambda i: (i, 0, 0)),
        pl.BlockSpec((1, NSUB, 1024), lambda i: (i, 0, 0)),
        pl.BlockSpec((1, NSUB, 1024), lambda i: (i, 0, 0)),
    ],
    out_specs=[pl.BlockSpec((1, 128), lambda i: (0, 0))] * _N_SUMS,
    out_shape=[jax.ShapeDtypeStruct((1, 128), jnp.float32)] * _N_SUMS,
)


def kernel(user, item, label, mask, users, items_control, items_treatment):
    user3d = user.reshape(NW, CPT, CH)
    item3d = item.reshape(NW, CPT, CH)
    pval4d = jnp.broadcast_to(
        jnp.arange(N, dtype=jnp.int32)[:, None], (N, PR)
    ).reshape(NW, CPT, CH, PR)
    item16_3d = (item * PR).reshape(NW, CPT, CH)

    _scatter_pos, _gather_one, _gather_it = _sc_kernels()
    pos = _scatter_pos(item3d, pval4d)

    users_rm = _transpose1(users.T)
    ue = _gather_one(user3d, users_rm)
    ic_rm = _transpose1(items_control.T)
    ice = _gather_one(item3d, ic_rm)
    it_rm = _transpose1(items_treatment.T)
    ite, qflat = _gather_it(
        item3d, item16_3d, pos.reshape(NUM_ITEMS * PR), it_rm
    )

    lab3 = label.reshape(N_BLK, NSUB, 1024)
    w3 = jnp.broadcast_to(
        mask.astype(jnp.float32)[:, None], (B, L)
    ).reshape(N_BLK, NSUB, 1024)
    q3 = qflat.reshape(N_BLK, NSUB, 1024)

    sums = _compute(ue, ice, ite, lab3, w3, q3)
    s_bce_c, s_bce_t, s_dc, s_dt, s_w, s_sw, s_win = (o[0, 0] for o in sums)

    seq_len = jnp.float32(L)
    cnt_t = s_w / seq_len
    cnt_c = jnp.float32(B) - cnt_t
    control_loss = s_bce_c / (cnt_c * seq_len)
    treatment_loss = s_bce_t / (cnt_t * seq_len)
    control_distance = s_dc / (cnt_c * seq_len)
    treatment_distance = s_dt / (cnt_t * seq_len)
    discrepancy_loss = s_sw / (s_win * jnp.float32(EMB))
    return (control_loss, treatment_loss, discrepancy_loss,
            control_distance, treatment_distance)
